# baseline (speedup 1.0000x reference)
import numpy as np
import jax
import jax.numpy as jnp
from jax.experimental import pallas as pl
from jax.experimental.pallas import tpu as pltpu


_NB = 1024


def _onehot_shift(n_out, n_pool, k, phase):
    h = np.arange(n_out)[:, None, None]
    s = np.arange(n_pool)[None, :, None]
    i = np.arange(k)[None, None, :]
    return (h == 2 * s + phase + i).astype(np.float32)


_A1 = [_onehot_shift(32, 14, 5, a) for a in range(2)]
_A2 = [_onehot_shift(14, 5, 5, a) for a in range(2)]


def _build_phase_weights(w_conv1, w_conv2):
    w1 = w_conv1.reshape(6, 25)
    w2 = w_conv2
    w1_j_ci = w1.reshape(6, 5, 5).transpose(2, 0, 1).reshape(5, 30)
    w2_j = w2.transpose(3, 1, 0, 2).reshape(5, 6 * 16 * 5)
    W1, W2 = [], []
    for a in range(2):
        for b in range(2):
            A1a = jnp.asarray(_A1[a].reshape(32 * 14, 5))
            B1b = jnp.asarray(_A1[b].reshape(32 * 14, 5))
            T = jnp.dot(B1b, w1_j_ci)
            T2 = T.reshape(448, 6, 5).reshape(448 * 6, 5)
            Z = jnp.dot(A1a, T2.T)
            m1 = Z.astype(jnp.bfloat16).reshape(32, 14, 32, 14, 6)
            W1.append(m1.transpose(0, 2, 1, 3, 4).reshape(1024, 1176))

            A2a = jnp.asarray(_A2[a].reshape(14 * 5, 5))
            B2b = jnp.asarray(_A2[b].reshape(14 * 5, 5))
            U = jnp.dot(B2b, w2_j)
            U2 = U.reshape(70 * 6 * 16, 5)
            Z2 = jnp.dot(A2a, U2.T)
            m2 = Z2.astype(jnp.bfloat16).reshape(14, 5, 14, 5, 6, 16)
            W2.append(m2.transpose(0, 2, 4, 5, 1, 3).reshape(1176, 400))
    return jnp.stack(W1), jnp.stack(W2)


def _lenet_block_kernel(x_ref, w1_ref, b1_ref, w2_ref, b2_ref,
                        wf1_ref, bf1_ref, wf2_ref, bf2_ref, wf3_ref, bf3_ref,
                        out_ref):
    f32 = jnp.float32
    x = x_ref[...]

    m = None
    for ph in range(4):
        y = jnp.dot(x, w1_ref[ph], preferred_element_type=f32)
        m = y if m is None else jnp.maximum(m, y)
    p1 = jnp.maximum(m + b1_ref[...], 0.0).astype(jnp.bfloat16)

    m2 = None
    for ph in range(4):
        y = jnp.dot(p1, w2_ref[ph], preferred_element_type=f32)
        m2 = y if m2 is None else jnp.maximum(m2, y)
    p2 = jnp.maximum(m2 + b2_ref[...], 0.0).astype(jnp.bfloat16)

    h1 = jnp.maximum(jnp.dot(p2, wf1_ref[...], preferred_element_type=f32)
                     + bf1_ref[...], 0.0).astype(jnp.bfloat16)
    h2 = jnp.maximum(jnp.dot(h1, wf2_ref[...], preferred_element_type=f32)
                     + bf2_ref[...], 0.0).astype(jnp.bfloat16)
    out_ref[...] = (jnp.dot(h2, wf3_ref[...], preferred_element_type=f32)
                    + bf3_ref[...])


@jax.jit
def kernel(x, w_conv1, b_conv1, w_conv2, b_conv2,
           w_fc1, b_fc1, w_fc2, b_fc2, w_fc3, b_fc3):
    B = x.shape[0]
    xb = x.reshape(B, 1024).astype(jnp.bfloat16)
    nb = _NB
    Bpad = ((B + nb - 1) // nb) * nb
    if Bpad != B:
        xb = jnp.pad(xb, ((0, Bpad - B), (0, 0)))

    W1, W2 = _build_phase_weights(w_conv1, w_conv2)
    b1row = jnp.tile(b_conv1, 196).reshape(1, 1176)
    b2row = jnp.repeat(b_conv2, 25).reshape(1, 400)

    out = pl.pallas_call(
        _lenet_block_kernel,
        out_shape=jax.ShapeDtypeStruct((Bpad, 10), jnp.float32),
        grid=(Bpad // nb,),
        in_specs=[
            pl.BlockSpec((nb, 1024), lambda i: (i, 0)),
            pl.BlockSpec((4, 1024, 1176), lambda i: (0, 0, 0)),
            pl.BlockSpec((1, 1176), lambda i: (0, 0)),
            pl.BlockSpec((4, 1176, 400), lambda i: (0, 0, 0)),
            pl.BlockSpec((1, 400), lambda i: (0, 0)),
            pl.BlockSpec((400, 120), lambda i: (0, 0)),
            pl.BlockSpec((1, 120), lambda i: (0, 0)),
            pl.BlockSpec((120, 84), lambda i: (0, 0)),
            pl.BlockSpec((1, 84), lambda i: (0, 0)),
            pl.BlockSpec((84, 10), lambda i: (0, 0)),
            pl.BlockSpec((1, 10), lambda i: (0, 0)),
        ],
        out_specs=pl.BlockSpec((nb, 10), lambda i: (i, 0)),
        compiler_params=pltpu.CompilerParams(
            dimension_semantics=("parallel",),
            vmem_limit_bytes=64 * 1024 * 1024,
        ),
    )(xb, W1, b1row, W2, b2row,
      w_fc1.astype(jnp.bfloat16), b_fc1,
      w_fc2.astype(jnp.bfloat16), b_fc2,
      w_fc3.astype(jnp.bfloat16), b_fc3)

    return out[:B]


# speedup vs baseline: 1.6045x; 1.6045x over previous
import numpy as np
import jax
import jax.numpy as jnp
from jax.experimental import pallas as pl
from jax.experimental.pallas import tpu as pltpu


_NB = 512


def _onehot_shift(n_out, n_pool, k, phase):
    h = np.arange(n_out)[:, None, None]
    s = np.arange(n_pool)[None, :, None]
    i = np.arange(k)[None, None, :]
    return (h == 2 * s + phase + i).astype(np.float32)


_A1 = [_onehot_shift(32, 14, 5, a) for a in range(2)]
_A2 = [_onehot_shift(14, 5, 5, a) for a in range(2)]


def _build_phase_weights(w_conv1, w_conv2):
    w1 = w_conv1.reshape(6, 5, 5)
    w2 = w_conv2
    W1, W2 = [], []
    for a in range(2):
        for b in range(2):
            A1a = jnp.asarray(_A1[a])
            B1b = jnp.asarray(_A1[b])
            m1 = jnp.einsum('hsi,wtj,cij->hwstc', A1a, B1b, w1)
            W1.append(m1.reshape(1024, 1176).astype(jnp.bfloat16))
            A2a = jnp.asarray(_A2[a])
            B2b = jnp.asarray(_A2[b])
            m2 = jnp.einsum('sui,tvj,ocij->stcouv', A2a, B2b, w2)
            W2.append(m2.reshape(1176, 400).astype(jnp.bfloat16))
    return W1, W2


def _lenet_block_kernel(x_ref,
                        w1a_ref, w1b_ref, w1c_ref, w1d_ref, b1_ref,
                        w2a_ref, w2b_ref, w2c_ref, w2d_ref, b2_ref,
                        wf1_ref, bf1_ref, wf2_ref, bf2_ref, wf3_ref, bf3_ref,
                        out_ref):
    f32 = jnp.float32
    x = x_ref[...]

    m = None
    for ref in (w1a_ref, w1b_ref, w1c_ref, w1d_ref):
        y = jnp.dot(x, ref[...], preferred_element_type=f32)
        m = y if m is None else jnp.maximum(m, y)
    p1 = jnp.maximum(m + b1_ref[...], 0.0).astype(jnp.bfloat16)

    m2 = None
    for ref in (w2a_ref, w2b_ref, w2c_ref, w2d_ref):
        y = jnp.dot(p1, ref[...], preferred_element_type=f32)
        m2 = y if m2 is None else jnp.maximum(m2, y)
    p2 = jnp.maximum(m2 + b2_ref[...], 0.0).astype(jnp.bfloat16)

    h1 = jnp.maximum(jnp.dot(p2, wf1_ref[...], preferred_element_type=f32)
                     + bf1_ref[...], 0.0).astype(jnp.bfloat16)
    h2 = jnp.maximum(jnp.dot(h1, wf2_ref[...], preferred_element_type=f32)
                     + bf2_ref[...], 0.0).astype(jnp.bfloat16)
    out_ref[...] = (jnp.dot(h2, wf3_ref[...], preferred_element_type=f32)
                    + bf3_ref[...])


@jax.jit
def kernel(x, w_conv1, b_conv1, w_conv2, b_conv2,
           w_fc1, b_fc1, w_fc2, b_fc2, w_fc3, b_fc3):
    B = x.shape[0]
    xb = x.reshape(B, 1024).astype(jnp.bfloat16)
    nb = _NB
    Bpad = ((B + nb - 1) // nb) * nb
    if Bpad != B:
        xb = jnp.pad(xb, ((0, Bpad - B), (0, 0)))

    W1, W2 = _build_phase_weights(w_conv1, w_conv2)
    b1row = jnp.tile(b_conv1, 196).reshape(1, 1176)
    b2row = jnp.repeat(b_conv2, 25).reshape(1, 400)

    full = lambda shape: pl.BlockSpec(shape, lambda i: tuple(0 for _ in shape))
    out = pl.pallas_call(
        _lenet_block_kernel,
        out_shape=jax.ShapeDtypeStruct((Bpad, 10), jnp.float32),
        grid=(Bpad // nb,),
        in_specs=[
            pl.BlockSpec((nb, 1024), lambda i: (i, 0)),
            full((1024, 1176)), full((1024, 1176)),
            full((1024, 1176)), full((1024, 1176)),
            full((1, 1176)),
            full((1176, 400)), full((1176, 400)),
            full((1176, 400)), full((1176, 400)),
            full((1, 400)),
            full((400, 120)), full((1, 120)),
            full((120, 84)), full((1, 84)),
            full((84, 10)), full((1, 10)),
        ],
        out_specs=pl.BlockSpec((nb, 10), lambda i: (i, 0)),
        compiler_params=pltpu.CompilerParams(
            dimension_semantics=("parallel",),
            vmem_limit_bytes=64 * 1024 * 1024,
        ),
    )(xb, *W1, b1row, *W2, b2row,
      w_fc1.astype(jnp.bfloat16), b_fc1,
      w_fc2.astype(jnp.bfloat16), b_fc2,
      w_fc3.astype(jnp.bfloat16), b_fc3)

    return out[:B]


# speedup vs baseline: 4.5194x; 2.8166x over previous
import numpy as np
import jax
import jax.numpy as jnp
from jax.experimental import pallas as pl
from jax.experimental.pallas import tpu as pltpu


_NB = 512


def _onehot_shift(n_out, n_pool, k, phase):
    h = np.arange(n_out)[:, None, None]
    s = np.arange(n_pool)[None, :, None]
    i = np.arange(k)[None, None, :]
    return (h == 2 * s + phase + i).astype(np.float32)


_B1 = [_onehot_shift(32, 14, 5, b) for b in range(2)]
_B2 = [_onehot_shift(14, 5, 5, b) for b in range(2)]


def _build_tiles(w_conv1, w_conv2):
    w1 = w_conv1.reshape(6, 5, 5)
    T = jnp.stack([
        jnp.einsum('wtj,cij->iwtc', jnp.asarray(_B1[b]), w1).reshape(5, 32, 84)
        for b in range(2)])
    E = jnp.stack([
        jnp.einsum('tvj,ocij->itcvo', jnp.asarray(_B2[b]), w_conv2)
        .reshape(5, 84, 80)
        for b in range(2)])
    return T.astype(jnp.bfloat16), E.astype(jnp.bfloat16)


def _builder_kernel(t_ref, e_ref, out1_ref, out2_ref):
    for a in range(2):
        for b in range(2):
            ab = a * 2 + b
            out1_ref[ab, :, :] = jnp.zeros((1024, 1176), jnp.bfloat16)
            out2_ref[ab, :, :] = jnp.zeros((1176, 400), jnp.bfloat16)
            for i in range(5):
                t_tile = t_ref[b, i]
                for s in range(14):
                    r = (2 * s + a + i) * 32
                    out1_ref[ab, r:r + 32, s * 84:(s + 1) * 84] = t_tile
                e_tile = e_ref[b, i]
                for u in range(5):
                    r = (2 * u + a + i) * 84
                    out2_ref[ab, r:r + 84, u * 80:(u + 1) * 80] = e_tile


def _build_phase_weights(w_conv1, w_conv2):
    T, E = _build_tiles(w_conv1, w_conv2)
    return pl.pallas_call(
        _builder_kernel,
        out_shape=(jax.ShapeDtypeStruct((4, 1024, 1176), jnp.bfloat16),
                   jax.ShapeDtypeStruct((4, 1176, 400), jnp.bfloat16)),
        grid=(1,),
        in_specs=[
            pl.BlockSpec((2, 5, 32, 84), lambda i: (0, 0, 0, 0)),
            pl.BlockSpec((2, 5, 84, 80), lambda i: (0, 0, 0, 0)),
        ],
        out_specs=(pl.BlockSpec((4, 1024, 1176), lambda i: (0, 0, 0)),
                   pl.BlockSpec((4, 1176, 400), lambda i: (0, 0, 0))),
        compiler_params=pltpu.CompilerParams(
            vmem_limit_bytes=64 * 1024 * 1024,
        ),
    )(T, E)


def _lenet_block_kernel(x_ref, w1_ref, b1_ref, w2_ref, b2_ref,
                        wf1_ref, bf1_ref, wf2_ref, bf2_ref, wf3_ref, bf3_ref,
                        out_ref):
    f32 = jnp.float32
    x = x_ref[...]

    m = None
    for ph in range(4):
        y = jnp.dot(x, w1_ref[ph], preferred_element_type=f32)
        m = y if m is None else jnp.maximum(m, y)
    p1 = jnp.maximum(m + b1_ref[...], 0.0).astype(jnp.bfloat16)

    m2 = None
    for ph in range(4):
        y = jnp.dot(p1, w2_ref[ph], preferred_element_type=f32)
        m2 = y if m2 is None else jnp.maximum(m2, y)
    p2 = jnp.maximum(m2 + b2_ref[...], 0.0).astype(jnp.bfloat16)

    h1 = jnp.maximum(jnp.dot(p2, wf1_ref[...], preferred_element_type=f32)
                     + bf1_ref[...], 0.0).astype(jnp.bfloat16)
    h2 = jnp.maximum(jnp.dot(h1, wf2_ref[...], preferred_element_type=f32)
                     + bf2_ref[...], 0.0).astype(jnp.bfloat16)
    out_ref[...] = (jnp.dot(h2, wf3_ref[...], preferred_element_type=f32)
                    + bf3_ref[...])


@jax.jit
def kernel(x, w_conv1, b_conv1, w_conv2, b_conv2,
           w_fc1, b_fc1, w_fc2, b_fc2, w_fc3, b_fc3):
    B = x.shape[0]
    xb = x.reshape(B, 1024).astype(jnp.bfloat16)
    nb = _NB
    Bpad = ((B + nb - 1) // nb) * nb
    if Bpad != B:
        xb = jnp.pad(xb, ((0, Bpad - B), (0, 0)))

    W1, W2 = _build_phase_weights(w_conv1, w_conv2)
    b1row = jnp.tile(b_conv1, 196).reshape(1, 1176)
    b2row = jnp.tile(b_conv2, 25).reshape(1, 400)
    wf1 = (w_fc1.reshape(16, 5, 5, 120).transpose(1, 2, 0, 3)
           .reshape(400, 120))

    full = lambda shape: pl.BlockSpec(shape, lambda i: tuple(0 for _ in shape))
    out = pl.pallas_call(
        _lenet_block_kernel,
        out_shape=jax.ShapeDtypeStruct((Bpad, 10), jnp.float32),
        grid=(Bpad // nb,),
        in_specs=[
            pl.BlockSpec((nb, 1024), lambda i: (i, 0)),
            full((4, 1024, 1176)),
            full((1, 1176)),
            full((4, 1176, 400)),
            full((1, 400)),
            full((400, 120)), full((1, 120)),
            full((120, 84)), full((1, 84)),
            full((84, 10)), full((1, 10)),
        ],
        out_specs=pl.BlockSpec((nb, 10), lambda i: (i, 0)),
        compiler_params=pltpu.CompilerParams(
            dimension_semantics=("parallel",),
            vmem_limit_bytes=64 * 1024 * 1024,
        ),
    )(xb, W1, b1row, W2, b2row,
      wf1.astype(jnp.bfloat16), b_fc1,
      w_fc2.astype(jnp.bfloat16), b_fc2,
      w_fc3.astype(jnp.bfloat16), b_fc3)

    return out[:B]


# speedup vs baseline: 4.6212x; 1.0225x over previous
import numpy as np
import jax
import jax.numpy as jnp
from jax.experimental import pallas as pl
from jax.experimental.pallas import tpu as pltpu


_NB = 512
_NP1 = 1280
_NP2 = 512


def _onehot_shift(n_out, n_pool, k, phase):
    h = np.arange(n_out)[:, None, None]
    s = np.arange(n_pool)[None, :, None]
    i = np.arange(k)[None, None, :]
    return (h == 2 * s + phase + i).astype(np.float32)


_B1 = [_onehot_shift(32, 14, 5, b) for b in range(2)]
_B2 = [_onehot_shift(14, 5, 5, b) for b in range(2)]


def _build_tiles(w_conv1, w_conv2):
    w1 = w_conv1.reshape(6, 5, 5)
    T = jnp.stack([
        jnp.einsum('wtj,cij->iwtc', jnp.asarray(_B1[b]), w1).reshape(5, 32, 84)
        for b in range(2)])
    E = jnp.stack([
        jnp.einsum('tvj,ocij->itcvo', jnp.asarray(_B2[b]), w_conv2)
        .reshape(5, 84, 80)
        for b in range(2)])
    return T.astype(jnp.bfloat16), E.astype(jnp.bfloat16)


def _builder_kernel(t_ref, e_ref, out1_ref, out2_ref):
    out1_ref[...] = jnp.zeros((1024, 4 * _NP1), jnp.bfloat16)
    out2_ref[...] = jnp.zeros((1176, 4 * _NP2), jnp.bfloat16)
    for a in range(2):
        for b in range(2):
            c0_1 = (a * 2 + b) * _NP1
            c0_2 = (a * 2 + b) * _NP2
            for i in range(5):
                t_tile = t_ref[b, i]
                for s in range(14):
                    r = (2 * s + a + i) * 32
                    out1_ref[r:r + 32, c0_1 + s * 84:c0_1 + (s + 1) * 84] = t_tile
                e_tile = e_ref[b, i]
                for u in range(5):
                    r = (2 * u + a + i) * 84
                    out2_ref[r:r + 84, c0_2 + u * 80:c0_2 + (u + 1) * 80] = e_tile


def _build_phase_weights(w_conv1, w_conv2):
    T, E = _build_tiles(w_conv1, w_conv2)
    return pl.pallas_call(
        _builder_kernel,
        out_shape=(jax.ShapeDtypeStruct((1024, 4 * _NP1), jnp.bfloat16),
                   jax.ShapeDtypeStruct((1176, 4 * _NP2), jnp.bfloat16)),
        grid=(1,),
        in_specs=[
            pl.BlockSpec((2, 5, 32, 84), lambda i: (0, 0, 0, 0)),
            pl.BlockSpec((2, 5, 84, 80), lambda i: (0, 0, 0, 0)),
        ],
        out_specs=(pl.BlockSpec((1024, 4 * _NP1), lambda i: (0, 0)),
                   pl.BlockSpec((1176, 4 * _NP2), lambda i: (0, 0))),
        compiler_params=pltpu.CompilerParams(
            vmem_limit_bytes=64 * 1024 * 1024,
        ),
    )(T, E)


def _lenet_block_kernel(x_ref, w1_ref, b1_ref, w2_ref, b2_ref,
                        wf1_ref, bf1_ref, wf2_ref, bf2_ref, wf3_ref, bf3_ref,
                        out_ref):
    f32 = jnp.float32
    x = x_ref[...].astype(jnp.bfloat16)

    y = jnp.dot(x, w1_ref[...], preferred_element_type=f32)
    m = jnp.maximum(
        jnp.maximum(y[:, 0 * _NP1:0 * _NP1 + 1176], y[:, 1 * _NP1:1 * _NP1 + 1176]),
        jnp.maximum(y[:, 2 * _NP1:2 * _NP1 + 1176], y[:, 3 * _NP1:3 * _NP1 + 1176]))
    p1 = jnp.maximum(m + b1_ref[...], 0.0).astype(jnp.bfloat16)

    y2 = jnp.dot(p1, w2_ref[...], preferred_element_type=f32)
    m2 = jnp.maximum(
        jnp.maximum(y2[:, 0 * _NP2:0 * _NP2 + 400], y2[:, 1 * _NP2:1 * _NP2 + 400]),
        jnp.maximum(y2[:, 2 * _NP2:2 * _NP2 + 400], y2[:, 3 * _NP2:3 * _NP2 + 400]))
    p2 = jnp.maximum(m2 + b2_ref[...], 0.0).astype(jnp.bfloat16)

    h1 = jnp.maximum(jnp.dot(p2, wf1_ref[...], preferred_element_type=f32)
                     + bf1_ref[...], 0.0).astype(jnp.bfloat16)
    h2 = jnp.maximum(jnp.dot(h1, wf2_ref[...], preferred_element_type=f32)
                     + bf2_ref[...], 0.0).astype(jnp.bfloat16)
    out_ref[...] = (jnp.dot(h2, wf3_ref[...], preferred_element_type=f32)
                    + bf3_ref[...])


@jax.jit
def kernel(x, w_conv1, b_conv1, w_conv2, b_conv2,
           w_fc1, b_fc1, w_fc2, b_fc2, w_fc3, b_fc3):
    B = x.shape[0]
    xb = x.reshape(B, 1024)
    nb = _NB
    Bpad = ((B + nb - 1) // nb) * nb
    if Bpad != B:
        xb = jnp.pad(xb, ((0, Bpad - B), (0, 0)))

    W1, W2 = _build_phase_weights(w_conv1, w_conv2)
    b1row = jnp.tile(b_conv1, 196).reshape(1, 1176)
    b2row = jnp.tile(b_conv2, 25).reshape(1, 400)
    wf1 = (w_fc1.reshape(16, 5, 5, 120).transpose(1, 2, 0, 3)
           .reshape(400, 120))

    full = lambda shape: pl.BlockSpec(shape, lambda i: tuple(0 for _ in shape))
    out = pl.pallas_call(
        _lenet_block_kernel,
        out_shape=jax.ShapeDtypeStruct((Bpad, 10), jnp.float32),
        grid=(Bpad // nb,),
        in_specs=[
            pl.BlockSpec((nb, 1024), lambda i: (i, 0)),
            full((1024, 4 * _NP1)),
            full((1, 1176)),
            full((1176, 4 * _NP2)),
            full((1, 400)),
            full((400, 120)), full((1, 120)),
            full((120, 84)), full((1, 84)),
            full((84, 10)), full((1, 10)),
        ],
        out_specs=pl.BlockSpec((nb, 10), lambda i: (i, 0)),
        compiler_params=pltpu.CompilerParams(
            dimension_semantics=("parallel",),
            vmem_limit_bytes=64 * 1024 * 1024,
        ),
    )(xb, W1, b1row, W2, b2row,
      wf1.astype(jnp.bfloat16), b_fc1,
      w_fc2.astype(jnp.bfloat16), b_fc2,
      w_fc3.astype(jnp.bfloat16), b_fc3)

    return out[:B]
